# baseline (speedup 1.0000x reference)
import functools

import jax
import jax.numpy as jnp
from jax import lax
from jax.experimental import pallas as pl
from jax.experimental.pallas import tpu as pltpu
from jax.experimental.pallas import tpu_sc as plsc

N = 10000
E = 320000
D = 128
H = 4
DH = D // H
W = D + 16

NC = 2
NS = 16
NW = NC * NS
EPW = E // NW
B = 32
NCHW = (E // B + 16) // NW
NPAD = 10240
ROWS_PT = NPAD // NS

RB = 1000


def _proj_body(x_ref, qw_ref, kw_ref, vw_ref, qo_ref, ko_ref, vo_ref):
    x = x_ref[...]
    qo_ref[...] = jnp.dot(x, qw_ref[...], preferred_element_type=jnp.float32)
    ko_ref[...] = jnp.dot(x, kw_ref[...], preferred_element_type=jnp.float32)
    vo_ref[...] = jnp.dot(x, vw_ref[...], preferred_element_type=jnp.float32)


def _project(embeds, qw, kw, vwp):
    row_spec = pl.BlockSpec((RB, D), lambda i: (i, 0))
    return pl.pallas_call(
        _proj_body,
        grid=(N // RB,),
        in_specs=[row_spec,
                  pl.BlockSpec((D, D), lambda i: (0, 0)),
                  pl.BlockSpec((D, D), lambda i: (0, 0)),
                  pl.BlockSpec((D, W), lambda i: (0, 0))],
        out_specs=[row_spec, row_spec, pl.BlockSpec((RB, W), lambda i: (i, 0))],
        out_shape=[jax.ShapeDtypeStruct((N, D), jnp.float32),
                   jax.ShapeDtypeStruct((N, D), jnp.float32),
                   jax.ShapeDtypeStruct((N, W), jnp.float32)],
    )(embeds, qw, kw, vwp)


def _edge_body(rc_hbm, qn_hbm, kn_hbm, vn_hbm, z_hbm,
               s_out,
               rcA, rcB, rcC, q0, k0, w0, q1, k1, w1, s_sh,
               sg0, sg1, sem2, sem3):
    cid = lax.axis_index("c")
    sid = lax.axis_index("s")

    pltpu.sync_copy(z_hbm.at[pl.ds(sid * ROWS_PT, ROWS_PT)],
                    s_sh.at[pl.ds(sid * ROWS_PT, ROWS_PT)])
    plsc.subcore_barrier()

    wstart = (cid * NS + sid) * NCHW
    lane = lax.iota(jnp.int32, 16)

    def compute(q_v, k_v, w_v):
        def body(i, _):
            h = jnp.bitwise_and(i, 3)
            eidx = lane + jnp.left_shift(jnp.right_shift(i, 2), 4)
            ebase = jnp.bitwise_and(eidx, 15)
            h8 = h * 8
            hDH = h * DH
            acc = [None] * 4
            for d in range(DH):
                col = jnp.bitwise_and(ebase + (h8 + d), DH - 1) + hDH
                qc = plsc.load_gather(q_v, [eidx, col])
                kc = plsc.load_gather(k_v, [eidx, col])
                p = qc * kc
                acc[d % 4] = p if acc[d % 4] is None else acc[d % 4] + p
            att = (acc[0] + acc[1]) + (acc[2] + acc[3])
            ea = jnp.exp(jnp.clip(att, -10.0, 10.0))
            plsc.store_scatter(w_v, [eidx, jnp.full((16,), D, jnp.int32) + h],
                               ea)
            for b in range(DH // 8):
                cols = [jnp.bitwise_and(ebase + (h8 + 5 + 8 * b + i2),
                                        DH - 1) + hDH
                        for i2 in range(8)]
                vals = [plsc.load_gather(w_v, [eidx, c]) for c in cols]
                for c, v in zip(cols, vals):
                    plsc.store_scatter(w_v, [eidx, c], v * ea)
            return 0
        lax.fori_loop(0, (B // 16) * H, body, 0)

    def gathers(rc, q_v, k_v, w_v, sg):
        pltpu.async_copy(qn_hbm.at[rc.at[0]], q_v, sg)
        pltpu.async_copy(kn_hbm.at[rc.at[1]], k_v, sg)
        pltpu.async_copy(vn_hbm.at[rc.at[1]], w_v, sg)

    def wait_gathers(rc, q_v, k_v, w_v, sg):
        pltpu.make_async_copy(qn_hbm.at[rc.at[0]], q_v, sg).wait()
        pltpu.make_async_copy(kn_hbm.at[rc.at[1]], k_v, sg).wait()
        pltpu.make_async_copy(vn_hbm.at[rc.at[1]], w_v, sg).wait()

    SETS = [(q0, k0, w0, sg0), (q1, k1, w1, sg1)]
    RCS = [rcA, rcB, rcC]

    pltpu.sync_copy(rc_hbm.at[wstart], rcA)
    pltpu.async_copy(rc_hbm.at[wstart + 1], rcB, sem3)
    pltpu.async_copy(z_hbm.at[pl.ds(0, B)], w1, sem2)
    gathers(rcA, q0, k0, w0, sg0)

    def step(c, cc):
        qc, kc, wc, sgc = SETS[cc & 1]
        qn_, kn_, wn_, sgn = SETS[(cc + 1) & 1]
        rc_cur = RCS[cc % 3]
        rc_nxt = RCS[(cc + 1) % 3]
        rc_n2 = RCS[(cc + 2) % 3]
        pltpu.make_async_copy(z_hbm.at[pl.ds(0, B)], wn_, sem2).wait()
        pltpu.make_async_copy(rc_hbm.at[0], rc_nxt, sem3).wait()
        pltpu.async_copy(rc_hbm.at[wstart + c + 2], rc_n2, sem3)
        gathers(rc_nxt, qn_, kn_, wn_, sgn)
        wait_gathers(rc_cur, qc, kc, wc, sgc)
        compute(qc, kc, wc)
        pltpu.async_copy(wc, s_sh.at[rc_cur.at[0]], sem2, add=True)

    def six_body(j, _):
        c0 = j * 6
        for t in range(6):
            step(c0 + t, t)
        return 0

    lax.fori_loop(0, NCHW // 6, six_body, 0)
    step(NCHW - 1, 0)

    pltpu.make_async_copy(z_hbm.at[pl.ds(0, B)], w0, sem2).wait()
    pltpu.make_async_copy(rc_hbm.at[0], rcC, sem3).wait()
    wait_gathers(rcB, q1, k1, w1, sg1)

    plsc.subcore_barrier()
    pltpu.sync_copy(s_sh.at[pl.ds(sid * ROWS_PT, ROWS_PT)],
                    s_out.at[cid, pl.ds(sid * ROWS_PT, ROWS_PT)])


_edge_kernel = functools.partial(
    pl.kernel,
    out_type=jax.ShapeDtypeStruct((NC, NPAD, W), jnp.float32),
    mesh=plsc.VectorSubcoreMesh(core_axis_name="c", subcore_axis_name="s"),
    compiler_params=pltpu.CompilerParams(needs_layout_passes=False,
                                         use_tc_tiling_on_sc=False),
    scratch_types=[
        pltpu.VMEM((2, B), jnp.int32),
        pltpu.VMEM((2, B), jnp.int32),
        pltpu.VMEM((2, B), jnp.int32),
        pltpu.VMEM((B, D), jnp.float32),
        pltpu.VMEM((B, D), jnp.float32),
        pltpu.VMEM((B, W), jnp.float32),
        pltpu.VMEM((B, D), jnp.float32),
        pltpu.VMEM((B, D), jnp.float32),
        pltpu.VMEM((B, W), jnp.float32),
        pltpu.VMEM_SHARED((NPAD, W), jnp.float32),
        pltpu.SemaphoreType.DMA,
        pltpu.SemaphoreType.DMA,
        pltpu.SemaphoreType.DMA,
        pltpu.SemaphoreType.DMA,
    ],
)(_edge_body)


def _combine_body(s_ref, emb_ref, m_ref, g_ref, b_ref, o_ref):
    s2 = s_ref[0, :, 0:D] + s_ref[1, :, 0:D]
    s1 = s_ref[0, :, D:W] + s_ref[1, :, D:W]
    den = jnp.dot(s1, m_ref[...], preferred_element_type=jnp.float32) + 1e-8
    res = s2 / den + emb_ref[...]
    mean = jnp.mean(res, axis=-1, keepdims=True)
    cen = res - mean
    var = jnp.mean(cen * cen, axis=-1, keepdims=True)
    o_ref[...] = cen * lax.rsqrt(var + 1e-6) * g_ref[...] + b_ref[...]


def _combine(sp, embeds, mexp, scale2d, bias2d):
    return pl.pallas_call(
        _combine_body,
        grid=(N // RB,),
        in_specs=[
            pl.BlockSpec((NC, RB, W), lambda i: (0, i, 0)),
            pl.BlockSpec((RB, D), lambda i: (i, 0)),
            pl.BlockSpec((W - D, D), lambda i: (0, 0)),
            pl.BlockSpec((1, D), lambda i: (0, 0)),
            pl.BlockSpec((1, D), lambda i: (0, 0)),
        ],
        out_specs=pl.BlockSpec((RB, D), lambda i: (i, 0)),
        out_shape=jax.ShapeDtypeStruct((N, D), jnp.float32),
    )(sp, embeds, mexp, scale2d, bias2d)


def kernel(embeds, edge_index, qTrans, kTrans, vTrans, ln_scale, ln_bias):
    rc = jnp.stack([edge_index[0].reshape(E // B, B),
                    edge_index[1].reshape(E // B, B)], axis=1)
    pad = jnp.concatenate([jnp.full((24, 1, B), NPAD - 1, jnp.int32),
                           jnp.zeros((24, 1, B), jnp.int32)], axis=1)
    rc = jnp.concatenate([rc, pad], axis=0)

    vwp = jnp.pad(vTrans, ((0, 0), (0, W - D)))
    qn, kn, vnp = _project(embeds, qTrans, kTrans, vwp)

    z = jnp.zeros((NPAD, W), jnp.float32)
    sp = _edge_kernel(rc, qn, kn, vnp, z)

    mexp = jnp.where(
        (jnp.arange(W - D, dtype=jnp.int32)[:, None]
         == jnp.arange(D, dtype=jnp.int32)[None, :] // DH),
        1.0, 0.0).astype(jnp.float32)

    return _combine(sp, embeds, mexp,
                    ln_scale.reshape(1, D), ln_bias.reshape(1, D))
